# baseline (speedup 1.0000x reference)
import functools

import jax
import jax.numpy as jnp
from jax import lax
from jax.experimental import pallas as pl
from jax.experimental.pallas import tpu as pltpu
from jax.experimental.pallas import tpu_sc as plsc

LANES = 16
NC, NS = 2, 16
NW = NC * NS
CHUNK = 128


def _transpose_body(wd_ref, ws_ref, td_ref, ts_ref):
    td_ref[...] = wd_ref[...].T
    ts_ref[...] = ws_ref[...].T


def _transpose_tables(W_drug, W_disease):
    emb, nd = W_drug.shape
    _, ns = W_disease.shape
    return pl.pallas_call(
        _transpose_body,
        out_shape=(
            jax.ShapeDtypeStruct((nd, emb), jnp.float32),
            jax.ShapeDtypeStruct((ns, emb), jnp.float32),
        ),
    )(W_drug, W_disease)


def _lane_splat(vec, i):
    idx = jnp.full((LANES,), i, dtype=jnp.int32)
    return lax.gather(
        vec,
        idx[:, None],
        dimension_numbers=lax.GatherDimensionNumbers(
            offset_dims=(), collapsed_slice_dims=(0,), start_index_map=(0,)
        ),
        slice_sizes=(1,),
        mode=lax.GatherScatterMode.PROMISE_IN_BOUNDS,
    )


def _newton_rsqrt(t):
    i = lax.bitcast_convert_type(t, jnp.int32)
    y = lax.bitcast_convert_type(jnp.int32(0x5F3759DF) - (i >> 1), jnp.float32)
    for _ in range(3):
        y = y * (1.5 - 0.5 * t * y * y)
    return y


def _make_sc_interact(B, EMB):
    BPW = B // NW
    NCHUNK = BPW // CHUNK
    NBLK = BPW // LANES
    JV = EMB // LANES

    QCH = BPW // 128
    mesh = plsc.VectorSubcoreMesh(core_axis_name="c", subcore_axis_name="s")

    @functools.partial(
        pl.kernel,
        mesh=mesh,
        compiler_params=pltpu.CompilerParams(
            needs_layout_passes=False, use_tc_tiling_on_sc=False),
        out_type=jax.ShapeDtypeStruct((B * EMB // 128, 128), jnp.float32),
        scratch_types=[
            pltpu.VMEM((BPW,), jnp.int32),
            pltpu.VMEM((BPW,), jnp.int32),
            pltpu.VMEM((BPW, EMB), jnp.float32),
            pltpu.VMEM((BPW, EMB), jnp.float32),
            pltpu.VMEM((BPW // LANES * LANES * 17,), jnp.float32),
            pltpu.VMEM((QCH * EMB, 129), jnp.float32),
            [pltpu.SemaphoreType.DMA] * 4,
            pltpu.SemaphoreType.DMA,
        ],
    )
    def sc_interact(a0_hbm, a1_hbm, tabd_hbm, tabs_hbm, out_hbm,
                    idx0_v, idx1_v, r0_v, r1_v, sqT_v, outT_v, gsems, osem):
        wid = lax.axis_index("s") * NC + lax.axis_index("c")
        base = wid * BPW

        iw0 = pltpu.async_copy(a0_hbm.at[pl.ds(base, BPW)], idx0_v, osem)
        iw1 = pltpu.async_copy(a1_hbm.at[pl.ds(base, BPW)], idx1_v, osem)
        iw0.wait()
        iw1.wait()

        gwaits = []
        for j in range(NCHUNK):
            sl = pl.ds(j * CHUNK, CHUNK)
            gwaits.append((
                pltpu.async_copy(tabd_hbm.at[idx0_v.at[sl]], r0_v.at[sl],
                                 gsems[j]),
                pltpu.async_copy(tabs_hbm.at[idx1_v.at[sl]], r1_v.at[sl],
                                 gsems[j]),
            ))

        iota = lax.iota(jnp.int32, LANES)
        iota17 = iota * 17
        voff = [iota + 16 * j for j in range(JV)]

        def block(blk):
            r = blk * LANES
            sqb = blk * (LANES * 17)
            for i in range(LANES):
                row = r + i
                s = jnp.zeros((LANES,), jnp.float32)
                for j in range(JV):
                    dd = pl.ds(j * LANES, LANES)
                    p = r0_v[row, dd] * r1_v[row, dd]
                    r0_v[row, dd] = p
                    s = s + p * p
                plsc.store_scatter(sqT_v, [iota17 + (sqb + i)], s)
            acc = sqT_v[pl.ds(sqb, LANES)]
            for l in range(1, LANES):
                acc = acc + sqT_v[pl.ds(sqb + l * 17, LANES)]
            y = _newton_rsqrt(jnp.maximum(acc, 1e-35))
            inv = 1.0 / jnp.maximum(acc * y, 1e-12)
            q = blk >> 3
            cblk = blk * LANES - q * 128
            bvecs = [voff[j] + q * EMB for j in range(JV)]
            for i in range(LANES):
                row = r + i
                g = _lane_splat(inv, i)
                cv = jnp.full((LANES,), cblk + i, jnp.int32)
                for j in range(JV):
                    dd = pl.ds(j * LANES, LANES)
                    plsc.store_scatter(outT_v, [bvecs[j], cv],
                                       r0_v[row, dd] * g)

        for ws in gwaits:
            for w in ws:
                w.wait()
        plsc.parallel_loop(0, NBLK, unroll=2)(block)
        owaits = []
        for qq in range(QCH):
            for ii in range(EMB // 8):
                src = (pl.ds(qq * EMB + ii * 8, 8), pl.ds(0, 128))
                dst = pl.ds((ii * (B // 128) + wid * QCH + qq) * 8, 8)
                owaits.append(pltpu.async_copy(
                    outT_v.at[src[0], src[1]], out_hbm.at[dst], osem))
        for w in owaits:
            w.wait()

    return sc_interact


def kernel(association_pairs, drug_embedding, disease_embedding, W_drug,
           W_disease):
    del drug_embedding, disease_embedding
    B = association_pairs.shape[1]
    EMB = W_drug.shape[0]
    tabd, tabs = _transpose_tables(W_drug, W_disease)
    a0 = association_pairs[0]
    a1 = association_pairs[1]
    scout = _make_sc_interact(B, EMB)(a0, a1, tabd, tabs)
    scout = scout.reshape(EMB // 8, B // 128, 8, 128)
    return scout.transpose((1, 3, 0, 2)).reshape(B, EMB)


# speedup vs baseline: 1.1642x; 1.1642x over previous
import functools

import jax
import jax.numpy as jnp
from jax import lax
from jax.experimental import pallas as pl
from jax.experimental.pallas import tpu as pltpu
from jax.experimental.pallas import tpu_sc as plsc

LANES = 16
NC, NS = 2, 16
NW = NC * NS
CHUNK = 128


def _transpose_body(wd_ref, ws_ref, td_ref, ts_ref):
    td_ref[...] = wd_ref[...].T
    ts_ref[...] = ws_ref[...].T


def _transpose_tables(W_drug, W_disease):
    emb, nd = W_drug.shape
    _, ns = W_disease.shape
    return pl.pallas_call(
        _transpose_body,
        out_shape=(
            jax.ShapeDtypeStruct((nd, emb), jnp.float32),
            jax.ShapeDtypeStruct((ns, emb), jnp.float32),
        ),
    )(W_drug, W_disease)


def _lane_splat(vec, i):
    idx = jnp.full((LANES,), i, dtype=jnp.int32)
    return lax.gather(
        vec,
        idx[:, None],
        dimension_numbers=lax.GatherDimensionNumbers(
            offset_dims=(), collapsed_slice_dims=(0,), start_index_map=(0,)
        ),
        slice_sizes=(1,),
        mode=lax.GatherScatterMode.PROMISE_IN_BOUNDS,
    )


def _newton_rsqrt(t):
    i = lax.bitcast_convert_type(t, jnp.int32)
    y = lax.bitcast_convert_type(jnp.int32(0x5F3759DF) - (i >> 1), jnp.float32)
    for _ in range(3):
        y = y * (1.5 - 0.5 * t * y * y)
    return y


def _make_sc_interact(B, EMB):
    BPW = B // NW
    NCHUNK = BPW // CHUNK
    NBLK = BPW // LANES
    JV = EMB // LANES

    QCH = BPW // 128
    mesh = plsc.VectorSubcoreMesh(core_axis_name="c", subcore_axis_name="s")

    @functools.partial(
        pl.kernel,
        mesh=mesh,
        compiler_params=pltpu.CompilerParams(
            needs_layout_passes=False, use_tc_tiling_on_sc=False),
        out_type=jax.ShapeDtypeStruct((B * EMB // 128, 128), jnp.float32),
        scratch_types=[
            pltpu.VMEM((BPW,), jnp.int32),
            pltpu.VMEM((BPW,), jnp.int32),
            pltpu.VMEM((BPW, EMB), jnp.float32),
            pltpu.VMEM((BPW, EMB), jnp.float32),
            pltpu.VMEM((BPW // LANES * LANES * 17,), jnp.float32),
            pltpu.VMEM((QCH * EMB, 129), jnp.float32),
            [pltpu.SemaphoreType.DMA] * 4,
            pltpu.SemaphoreType.DMA,
        ],
    )
    def sc_interact(a0_hbm, a1_hbm, tabd_hbm, tabs_hbm, out_hbm,
                    idx0_v, idx1_v, r0_v, r1_v, sqT_v, outT_v, gsems, osem):
        wid = lax.axis_index("s") * NC + lax.axis_index("c")
        base = wid * BPW

        iw0 = pltpu.async_copy(a0_hbm.at[pl.ds(base, BPW)], idx0_v, osem)
        iw1 = pltpu.async_copy(a1_hbm.at[pl.ds(base, BPW)], idx1_v, osem)
        iw0.wait()
        iw1.wait()

        gwaits = []
        for j in range(NCHUNK):
            sl = pl.ds(j * CHUNK, CHUNK)
            gwaits.append((
                pltpu.async_copy(tabd_hbm.at[idx0_v.at[sl]], r0_v.at[sl],
                                 gsems[j]),
                pltpu.async_copy(tabs_hbm.at[idx1_v.at[sl]], r1_v.at[sl],
                                 gsems[j]),
            ))

        iota = lax.iota(jnp.int32, LANES)
        iota17 = iota * 17
        voff = [iota + 16 * j for j in range(JV)]

        def block(blk):
            r = blk * LANES
            sqb = blk * (LANES * 17)
            for i in range(LANES):
                row = r + i
                s = jnp.zeros((LANES,), jnp.float32)
                for j in range(JV):
                    dd = pl.ds(j * LANES, LANES)
                    p = r0_v[row, dd] * r1_v[row, dd]
                    r0_v[row, dd] = p
                    s = s + p * p
                plsc.store_scatter(sqT_v, [iota17 + (sqb + i)], s)
            acc = sqT_v[pl.ds(sqb, LANES)]
            for l in range(1, LANES):
                acc = acc + sqT_v[pl.ds(sqb + l * 17, LANES)]
            y = _newton_rsqrt(jnp.maximum(acc, 1e-35))
            inv = 1.0 / jnp.maximum(acc * y, 1e-12)
            q = blk >> 3
            cblk = blk * LANES - q * 128
            bvecs = [voff[j] + q * EMB for j in range(JV)]
            for i in range(LANES):
                row = r + i
                g = _lane_splat(inv, i)
                cv = jnp.full((LANES,), cblk + i, jnp.int32)
                for j in range(JV):
                    dd = pl.ds(j * LANES, LANES)
                    plsc.store_scatter(outT_v, [bvecs[j], cv],
                                       r0_v[row, dd] * g)

        for ws in gwaits:
            for w in ws:
                w.wait()
        plsc.parallel_loop(0, NBLK)(block)
        owaits = []
        for qq in range(QCH):
            for ii in range(EMB // 8):
                src = (pl.ds(qq * EMB + ii * 8, 8), pl.ds(0, 128))
                dst = pl.ds((ii * (B // 128) + wid * QCH + qq) * 8, 8)
                owaits.append(pltpu.async_copy(
                    outT_v.at[src[0], src[1]], out_hbm.at[dst], osem))
        for w in owaits:
            w.wait()

    return sc_interact


def kernel(association_pairs, drug_embedding, disease_embedding, W_drug,
           W_disease):
    del drug_embedding, disease_embedding
    B = association_pairs.shape[1]
    EMB = W_drug.shape[0]
    tabd, tabs = _transpose_tables(W_drug, W_disease)
    a0 = association_pairs[0]
    a1 = association_pairs[1]
    scout = _make_sc_interact(B, EMB)(a0, a1, tabd, tabs)
    scout = scout.reshape(EMB // 8, B // 128, 8, 128)
    return scout.transpose((1, 3, 0, 2)).reshape(B, EMB)
